# baseline (speedup 1.0000x reference)
import jax
import jax.numpy as jnp
from jax.experimental import pallas as pl

B, L, D, BIN = 4096, 200, 64, 12
BIN_ALPHA = 1.0
PAD_TOKEN_ID = 0.0

_BB = 8192


def _body(x_ref, w1_ref, b1_ref, w2_ref, b2_ref, emb_ref, pad_ref, o_ref):
    x = x_ref[...]
    h = x * w1_ref[...] + b1_ref[...]
    h = jnp.maximum(h, 0.1 * h)
    h2 = jax.lax.dot_general(h, w2_ref[...], (((1,), (0,)), ((), ())),
                             preferred_element_type=jnp.float32)
    logits = BIN_ALPHA * h + h2 + b2_ref[...]
    m = jnp.max(logits, axis=-1, keepdims=True)
    e = jnp.exp(logits - m)
    w = e / jnp.sum(e, axis=-1, keepdims=True)
    out = jax.lax.dot_general(w, emb_ref[...], (((1,), (0,)), ((), ())),
                              preferred_element_type=jnp.float32)
    out = jnp.where(x == PAD_TOKEN_ID, pad_ref[...], out)
    o_ref[...] = out


def kernel(x, w1, b1, w2, b2, emb, emb_pad):
    n = B * L
    x_col = x.reshape(n, 1)
    small = pl.BlockSpec(index_map=lambda i: (0, 0))
    out = pl.pallas_call(
        _body,
        grid=(n // _BB,),
        in_specs=[
            pl.BlockSpec((_BB, 1), index_map=lambda i: (i, 0)),
            small, small, small, small, small, small,
        ],
        out_specs=pl.BlockSpec((_BB, D), index_map=lambda i: (i, 0)),
        out_shape=jax.ShapeDtypeStruct((n, D), jnp.float32),
    )(x_col, w1, b1.reshape(1, BIN), w2, b2.reshape(1, BIN), emb, emb_pad)
    return out.reshape(B, L, D)


# speedup vs baseline: 1.8244x; 1.8244x over previous
import jax
import jax.numpy as jnp
from jax.experimental import pallas as pl

B, L, D, BIN = 4096, 200, 64, 12
BIN_ALPHA = 1.0
PAD_TOKEN_ID = 0.0

_NB = 4096


def _body(x_ref, w1_ref, b1_ref, w2_ref, b2_ref, emb_ref, pad_ref, o_ref):
    x = x_ref[...].reshape(1, _NB)
    w1c = w1_ref[...].reshape(BIN, 1)
    b1c = b1_ref[...].reshape(BIN, 1)
    b2c = b2_ref[...].reshape(BIN, 1)
    h = x * w1c + b1c
    h = jnp.maximum(h, 0.1 * h)
    h2 = jax.lax.dot_general(w2_ref[...], h, (((0,), (0,)), ((), ())),
                             preferred_element_type=jnp.float32)
    logits = BIN_ALPHA * h + h2 + b2c
    m = jnp.max(logits, axis=0, keepdims=True)
    e = jnp.exp(logits - m)
    w = e * (1.0 / jnp.sum(e, axis=0, keepdims=True))
    pad = (x == PAD_TOKEN_ID)
    w13 = jnp.concatenate([jnp.where(pad, 0.0, w),
                           jnp.where(pad, 1.0, jnp.zeros_like(x))], axis=0)
    emb13 = jnp.concatenate([emb_ref[...], pad_ref[...]], axis=0)
    o_ref[...] = jax.lax.dot_general(w13, emb13, (((0,), (0,)), ((), ())),
                                     preferred_element_type=jnp.float32)


def kernel(x, w1, b1, w2, b2, emb, emb_pad):
    n = B * L
    x_rows = x.reshape(n // _NB, 1, _NB)
    small = pl.BlockSpec(index_map=lambda i: (0, 0))
    out = pl.pallas_call(
        _body,
        grid=(n // _NB,),
        in_specs=[
            pl.BlockSpec((1, 1, _NB), index_map=lambda i: (i, 0, 0)),
            small, small, small, small, small, small,
        ],
        out_specs=pl.BlockSpec((_NB, D), index_map=lambda i: (i, 0)),
        out_shape=jax.ShapeDtypeStruct((n, D), jnp.float32),
    )(x_rows, w1, b1.reshape(1, BIN), w2, b2.reshape(1, BIN), emb, emb_pad)
    return out.reshape(B, L, D)


# speedup vs baseline: 2.2342x; 1.2246x over previous
import jax
import jax.numpy as jnp
from jax.experimental import pallas as pl

B, L, D, BIN = 4096, 200, 64, 12
BIN_ALPHA = 1.0
PAD_TOKEN_ID = 0.0

_NB = 16384


def _body(x_ref, w1_ref, b1_ref, w2_ref, b2_ref, emb_ref, pad_ref, o_ref):
    x = x_ref[...].reshape(1, _NB)
    w1c = w1_ref[...].reshape(BIN, 1)
    b1c = b1_ref[...].reshape(BIN, 1)
    b2c = b2_ref[...].reshape(BIN, 1)
    h = x * w1c + b1c
    h = jnp.maximum(h, 0.1 * h)
    h2 = jax.lax.dot_general(w2_ref[...], h, (((0,), (0,)), ((), ())),
                             preferred_element_type=jnp.float32)
    logits = BIN_ALPHA * h + h2 + b2c
    m = jnp.max(logits, axis=0, keepdims=True)
    e = jnp.exp(logits - m)
    w = e * (1.0 / jnp.sum(e, axis=0, keepdims=True))
    pad = (x == PAD_TOKEN_ID)
    w13 = jnp.concatenate([jnp.where(pad, 0.0, w),
                           jnp.where(pad, 1.0, jnp.zeros_like(x))], axis=0)
    emb13 = jnp.concatenate([emb_ref[...], pad_ref[...]], axis=0)
    o_ref[...] = jax.lax.dot_general(w13, emb13, (((0,), (0,)), ((), ())),
                                     preferred_element_type=jnp.float32)


def kernel(x, w1, b1, w2, b2, emb, emb_pad):
    n = B * L
    x_rows = x.reshape(n // _NB, 1, _NB)
    small = pl.BlockSpec(index_map=lambda i: (0, 0))
    out = pl.pallas_call(
        _body,
        grid=(n // _NB,),
        in_specs=[
            pl.BlockSpec((1, 1, _NB), index_map=lambda i: (i, 0, 0)),
            small, small, small, small, small, small,
        ],
        out_specs=pl.BlockSpec((_NB, D), index_map=lambda i: (i, 0)),
        out_shape=jax.ShapeDtypeStruct((n, D), jnp.float32),
    )(x_rows, w1, b1.reshape(1, BIN), w2, b2.reshape(1, BIN), emb, emb_pad)
    return out.reshape(B, L, D)
